# baseline (speedup 1.0000x reference)
import functools

import jax
import jax.numpy as jnp
from jax import lax
from jax.experimental import pallas as pl
from jax.experimental.pallas import tpu as pltpu
from jax.experimental.pallas import tpu_sc as plsc

N = 10000
E = 160000
D = 128
NET = 3

NC = 2
NS = 16
NW = NC * NS

CH = 125
NCHUNK = E // CH
CT = NCHUNK // NW
CORE_ROWS = NCHUNK // NC

NPAD = 10112
ROWS_PER_TILE = NPAD // NS

NSLOT = 10240
NDEGC = 2 * NET * NSLOT
DEGC_PER_TILE = NDEGC // NS
PACK_N = NSLOT // NS
DCHUNK = 2 * NET * NCHUNK
DCT = DCHUNK // NW

BT = 1000
GRID = N // BT

_mesh = plsc.VectorSubcoreMesh(core_axis_name="c", subcore_axis_name="s")


@functools.partial(
    pl.kernel,
    mesh=_mesh,
    out_type=jax.ShapeDtypeStruct((NC * NSLOT * D,), jnp.float32),
    scratch_types=[
        pltpu.VMEM((DCT, CH), jnp.int32),
        pltpu.VMEM((CH,), jnp.float32),
        pltpu.VMEM((PACK_N,), jnp.float32),
        pltpu.VMEM((PACK_N,), jnp.int32),
        pltpu.VMEM_SHARED((NDEGC,), jnp.float32),
        pltpu.VMEM_SHARED((NSLOT * D,), jnp.float32),
    ],
)
def _sc_degrees(idx6_h, pidx_h, ones_h, zeros_h, out_h,
                idx_v, ones_v, cstage_v, pidx_v, degc_sh, pack_sh):
    c = lax.axis_index("c")
    s = lax.axis_index("s")
    pltpu.sync_copy(ones_h, ones_v)
    pltpu.sync_copy(idx6_h.at[pl.ds((c * NS + s) * DCT, DCT)], idx_v)
    r0 = s * DEGC_PER_TILE
    pltpu.sync_copy(zeros_h.at[pl.ds(r0, DEGC_PER_TILE)],
                    degc_sh.at[pl.ds(r0, DEGC_PER_TILE)])
    plsc.subcore_barrier()

    def step(t, carry):
        pltpu.sync_copy(ones_v, degc_sh.at[idx_v.at[t]], add=True)
        return carry

    lax.fori_loop(0, DCT, step, None)
    plsc.subcore_barrier()
    for d in range(2 * NET):
        pltpu.sync_copy(degc_sh.at[pl.ds(d * NSLOT + s * PACK_N, PACK_N)],
                        cstage_v)
        pltpu.sync_copy(pidx_h.at[d, 0, pl.ds(s * PACK_N, PACK_N)], pidx_v)
        pltpu.sync_copy(cstage_v, pack_sh.at[pidx_v])
    plsc.subcore_barrier()
    r1 = s * PACK_N * D
    pltpu.sync_copy(pack_sh.at[pl.ds(r1, PACK_N * D)],
                    out_h.at[pl.ds(c * NSLOT * D + r1, PACK_N * D)])


@functools.partial(
    pl.kernel,
    mesh=_mesh,
    out_type=jax.ShapeDtypeStruct((NET, NC, NPAD, D), jnp.float32),
    scratch_types=[
        pltpu.VMEM((CT, CH), jnp.int32),
        pltpu.VMEM((CT, CH), jnp.int32),
        pltpu.VMEM((CH, D), jnp.float32),
        pltpu.VMEM((CH, D), jnp.float32),
        pltpu.VMEM_SHARED((NPAD, D), jnp.float32),
        pltpu.SemaphoreType.DMA,
        pltpu.SemaphoreType.DMA,
    ],
)
def _sc_agg(tab0_h, tab1_h, tab2_h, ei0_h, ei1_h, ei2_h, zeros_h, out_h,
            idxs_v, idxd_v, rows0_v, rows1_v, agg_sh, sem0, sem1):
    c = lax.axis_index("c")
    s = lax.axis_index("s")
    w0 = (c * NS + s) * CT
    r0 = s * ROWS_PER_TILE
    tabs = (tab0_h, tab1_h, tab2_h)
    eis = (ei0_h, ei1_h, ei2_h)
    for e in range(NET):
        tab_h = tabs[e]
        pltpu.sync_copy(eis[e].at[0, pl.ds(w0, CT)], idxs_v)
        pltpu.sync_copy(eis[e].at[1, pl.ds(w0, CT)], idxd_v)
        pltpu.sync_copy(zeros_h.at[pl.ds(r0, ROWS_PER_TILE)],
                        agg_sh.at[pl.ds(r0, ROWS_PER_TILE)])
        plsc.subcore_barrier()
        b = 0
        pltpu.async_copy(tab_h.at[idxs_v.at[b]], rows0_v, sem0)
        pltpu.async_copy(tab_h.at[idxs_v.at[b + 1]], rows1_v, sem1)

        def step(t2, carry):
            t = b + 2 * t2
            pltpu.make_async_copy(tab_h.at[idxs_v.at[t]], rows0_v, sem0).wait()
            pltpu.sync_copy(rows0_v, agg_sh.at[idxd_v.at[t]], add=True)
            pltpu.async_copy(tab_h.at[idxs_v.at[t + 2]], rows0_v, sem0)
            pltpu.make_async_copy(
                tab_h.at[idxs_v.at[t + 1]], rows1_v, sem1).wait()
            pltpu.sync_copy(rows1_v, agg_sh.at[idxd_v.at[t + 1]], add=True)
            pltpu.async_copy(tab_h.at[idxs_v.at[t + 3]], rows1_v, sem1)
            return carry

        lax.fori_loop(0, CT // 2 - 1, step, None)
        t = b + CT - 2
        pltpu.make_async_copy(tab_h.at[idxs_v.at[t]], rows0_v, sem0).wait()
        pltpu.sync_copy(rows0_v, agg_sh.at[idxd_v.at[t]], add=True)
        pltpu.make_async_copy(tab_h.at[idxs_v.at[t + 1]], rows1_v, sem1).wait()
        pltpu.sync_copy(rows1_v, agg_sh.at[idxd_v.at[t + 1]], add=True)
        plsc.subcore_barrier()
        pltpu.sync_copy(agg_sh.at[pl.ds(r0, ROWS_PER_TILE)],
                        out_h.at[e, c, pl.ds(r0, ROWS_PER_TILE)])


def _rs_block(degw):
    return lax.rsqrt(jnp.maximum(degw[0] + degw[1], 1.0))


def _prep_body(x_ref, degw_ref, t0_ref, t1_ref, t2_ref):
    rs = _rs_block(degw_ref[...])
    x = x_ref[...]
    for e, tref in enumerate((t0_ref, t1_ref, t2_ref)):
        tref[...] = x * rs[:, 2 * e:2 * e + 1]


def _prep(x, degw):
    return pl.pallas_call(
        _prep_body,
        grid=(GRID,),
        in_specs=[
            pl.BlockSpec((BT, D), lambda b: (b, 0)),
            pl.BlockSpec((NC, BT, D), lambda b: (0, b, 0)),
        ],
        out_specs=[pl.BlockSpec((BT, D), lambda b: (b, 0))] * NET,
        out_shape=[jax.ShapeDtypeStruct((N, D), jnp.float32)] * NET,
    )(x, degw)


def _layer_a_body(aggs_ref, degw_ref, w3_ref, bsum_ref,
                  fcw_ref, fcb_ref, u_ref, sums_ref):
    b = pl.program_id(0)
    rs = _rs_block(degw_ref[...])
    tot = bsum_ref[...] + jnp.zeros((BT, D), jnp.float32)
    for e in range(NET):
        p = (aggs_ref[e, 0] + aggs_ref[e, 1]) * rs[:, 2 * e + 1:2 * e + 2]
        tot = tot + jnp.dot(p, w3_ref[e], preferred_element_type=jnp.float32)
    u = jnp.dot(tot, fcw_ref[...], preferred_element_type=jnp.float32)
    u = jnp.maximum(u + fcb_ref[...], 0.0)
    u_ref[...] = u

    @pl.when(b == 0)
    def _():
        sums_ref[...] = jnp.zeros_like(sums_ref)

    sums_ref[0:1] = sums_ref[0:1] + jnp.sum(u, axis=0, keepdims=True)
    sums_ref[1:2] = sums_ref[1:2] + jnp.sum(u * u, axis=0, keepdims=True)


def _layer_a(aggs, degw, w3, bsum, fcw, fcb):
    return pl.pallas_call(
        _layer_a_body,
        grid=(GRID,),
        in_specs=[
            pl.BlockSpec((NET, NC, BT, D), lambda b: (0, 0, b, 0)),
            pl.BlockSpec((NC, BT, D), lambda b: (0, b, 0)),
            pl.BlockSpec((NET, D, D), lambda b: (0, 0, 0)),
            pl.BlockSpec((1, D), lambda b: (0, 0)),
            pl.BlockSpec((D, D), lambda b: (0, 0)),
            pl.BlockSpec((1, D), lambda b: (0, 0)),
        ],
        out_specs=[
            pl.BlockSpec((BT, D), lambda b: (b, 0)),
            pl.BlockSpec((2, D), lambda b: (0, 0)),
        ],
        out_shape=[
            jax.ShapeDtypeStruct((N, D), jnp.float32),
            jax.ShapeDtypeStruct((2, D), jnp.float32),
        ],
    )(aggs, degw, w3, bsum, fcw, fcb)


def _norm(u_ref, sums_ref, gb_ref):
    mean = sums_ref[0:1] / N
    var = sums_ref[1:2] / N - mean * mean
    inv = lax.rsqrt(var + 1e-5)
    return (u_ref[...] - mean) * (inv * gb_ref[0:1]) + gb_ref[1:2]


def _layer_b_mid_body(u_ref, sums_ref, gb_ref, degw_ref,
                      t0_ref, t1_ref, t2_ref):
    hn = _norm(u_ref, sums_ref, gb_ref)
    rs = _rs_block(degw_ref[...])
    for e, tref in enumerate((t0_ref, t1_ref, t2_ref)):
        tref[...] = hn * rs[:, 2 * e:2 * e + 1]


def _layer_b_mid(u, sums, gb, degw):
    return pl.pallas_call(
        _layer_b_mid_body,
        grid=(GRID,),
        in_specs=[
            pl.BlockSpec((BT, D), lambda b: (b, 0)),
            pl.BlockSpec((2, D), lambda b: (0, 0)),
            pl.BlockSpec((2, D), lambda b: (0, 0)),
            pl.BlockSpec((NC, BT, D), lambda b: (0, b, 0)),
        ],
        out_specs=[pl.BlockSpec((BT, D), lambda b: (b, 0))] * NET,
        out_shape=[jax.ShapeDtypeStruct((N, D), jnp.float32)] * NET,
    )(u, sums, gb, degw)


def _layer_b_fin_body(u_ref, sums_ref, gb_ref, h_ref):
    h_ref[...] = _norm(u_ref, sums_ref, gb_ref)


def _layer_b_fin(u, sums, gb):
    return pl.pallas_call(
        _layer_b_fin_body,
        grid=(GRID,),
        in_specs=[
            pl.BlockSpec((BT, D), lambda b: (b, 0)),
            pl.BlockSpec((2, D), lambda b: (0, 0)),
            pl.BlockSpec((2, D), lambda b: (0, 0)),
        ],
        out_specs=pl.BlockSpec((BT, D), lambda b: (b, 0)),
        out_shape=jax.ShapeDtypeStruct((N, D), jnp.float32),
    )(u, sums, gb)


def kernel(x, params, edge_index_residue, edge_index_seq, edge_index_knn):
    eis = [ei.astype(jnp.int32) for ei in
           (edge_index_residue, edge_index_seq, edge_index_knn)]
    eic = [ei.reshape(2, NCHUNK, CH) for ei in eis]

    idx6 = jnp.concatenate(
        [jnp.concatenate([eis[e][0] + (2 * e) * NSLOT,
                          eis[e][1] + (2 * e + 1) * NSLOT])
         for e in range(NET)]
    ).reshape(DCHUNK, CH)

    ones_h = jnp.ones((CH,), jnp.float32)
    zeros_deg = jnp.zeros((NDEGC,), jnp.float32)
    pidx = ((jnp.arange(NSLOT, dtype=jnp.int32) * D)[None, :]
            + jnp.arange(2 * NET, dtype=jnp.int32)[:, None]
            ).reshape(2 * NET, 1, NSLOT)
    zeros_agg = jnp.zeros((NPAD, D), jnp.float32)

    degw = _sc_degrees(idx6, pidx, ones_h, zeros_deg).reshape(NC, NSLOT, D)
    tabs = _prep(x, degw)

    h = None
    for l, p in enumerate(params):
        w3 = jnp.stack([p['W_residue'], p['W_seq'], p['W_knn']])
        bsum = (p['b_residue'] + p['b_seq'] + p['b_knn']).reshape(1, D)
        fcb = p['fcb'].reshape(1, D)
        gb = jnp.stack([p['gamma'], p['beta']])
        aggs = _sc_agg(tabs[0], tabs[1], tabs[2],
                       eic[0], eic[1], eic[2], zeros_agg)
        u, sums = _layer_a(aggs, degw, w3, bsum, p['fcW'], fcb)
        if l + 1 < len(params):
            tabs = _layer_b_mid(u, sums, gb, degw)
        else:
            h = _layer_b_fin(u, sums, gb)
    return h


# speedup vs baseline: 1.0549x; 1.0549x over previous
import functools

import jax
import jax.numpy as jnp
from jax import lax
from jax.experimental import pallas as pl
from jax.experimental.pallas import tpu as pltpu
from jax.experimental.pallas import tpu_sc as plsc

N = 10000
E = 160000
D = 128
NET = 3

NC = 2
NS = 16
NW = NC * NS

CH = 125
NCHUNK = E // CH
CT = NCHUNK // NW
CORE_ROWS = NCHUNK // NC

NPAD = 10112
ROWS_PER_TILE = NPAD // NS

NSLOT = 10240
NDEGC = 2 * NET * NSLOT
DEGC_PER_TILE = NDEGC // NS
PACK_N = NSLOT // NS
DCHUNK = 2 * NET * NCHUNK
DCT = DCHUNK // NW

BT = 1000
GRID = N // BT

_mesh = plsc.VectorSubcoreMesh(core_axis_name="c", subcore_axis_name="s")


@functools.partial(
    pl.kernel,
    mesh=_mesh,
    out_type=jax.ShapeDtypeStruct((NC * NSLOT * D,), jnp.float32),
    scratch_types=[
        pltpu.VMEM((CT, CH), jnp.int32),
        pltpu.VMEM((CH,), jnp.float32),
        pltpu.VMEM((PACK_N,), jnp.float32),
        pltpu.VMEM((PACK_N,), jnp.int32),
        pltpu.VMEM_SHARED((NSLOT,), jnp.float32),
        pltpu.VMEM_SHARED((NSLOT,), jnp.float32),
        pltpu.VMEM_SHARED((NSLOT,), jnp.float32),
        pltpu.VMEM_SHARED((NSLOT,), jnp.float32),
        pltpu.VMEM_SHARED((NSLOT,), jnp.float32),
        pltpu.VMEM_SHARED((NSLOT,), jnp.float32),
        pltpu.VMEM_SHARED((NSLOT * D,), jnp.float32),
    ],
)
def _sc_degrees(ei0_h, ei1_h, ei2_h, pidx_h, ones_h, zeros_h, out_h,
                idx_v, ones_v, cstage_v, pidx_v,
                dg0, dg1, dg2, dg3, dg4, dg5, pack_sh):
    c = lax.axis_index("c")
    s = lax.axis_index("s")
    eis = (ei0_h, ei1_h, ei2_h)
    degs = (dg0, dg1, dg2, dg3, dg4, dg5)
    w0 = (c * NS + s) * CT
    z0 = s * PACK_N
    pltpu.sync_copy(ones_h, ones_v)
    for d6 in range(2 * NET):
        pltpu.sync_copy(zeros_h.at[pl.ds(z0, PACK_N)],
                        degs[d6].at[pl.ds(z0, PACK_N)])
    plsc.subcore_barrier()
    for d6 in range(2 * NET):
        e, dr = divmod(d6, 2)
        pltpu.sync_copy(eis[e].at[dr, pl.ds(w0, CT)], idx_v)
        deg = degs[d6]

        def step(t, carry):
            pltpu.sync_copy(ones_v, deg.at[idx_v.at[t]], add=True)
            return carry

        lax.fori_loop(0, CT, step, None)
    plsc.subcore_barrier()
    for d6 in range(2 * NET):
        pltpu.sync_copy(degs[d6].at[pl.ds(z0, PACK_N)], cstage_v)
        pltpu.sync_copy(pidx_h.at[d6, 0, pl.ds(z0, PACK_N)], pidx_v)
        pltpu.sync_copy(cstage_v, pack_sh.at[pidx_v])
    plsc.subcore_barrier()
    r1 = s * PACK_N * D
    pltpu.sync_copy(pack_sh.at[pl.ds(r1, PACK_N * D)],
                    out_h.at[pl.ds(c * NSLOT * D + r1, PACK_N * D)])


@functools.partial(
    pl.kernel,
    mesh=_mesh,
    out_type=jax.ShapeDtypeStruct((NET, NC, NPAD, D), jnp.float32),
    scratch_types=[
        pltpu.VMEM((CT, CH), jnp.int32),
        pltpu.VMEM((CT, CH), jnp.int32),
        pltpu.VMEM((CH, D), jnp.float32),
        pltpu.VMEM((CH, D), jnp.float32),
        pltpu.VMEM_SHARED((NPAD, D), jnp.float32),
        pltpu.SemaphoreType.DMA,
        pltpu.SemaphoreType.DMA,
    ],
)
def _sc_agg(tab0_h, tab1_h, tab2_h, ei0_h, ei1_h, ei2_h, zeros_h, out_h,
            idxs_v, idxd_v, rows0_v, rows1_v, agg_sh, sem0, sem1):
    c = lax.axis_index("c")
    s = lax.axis_index("s")
    w0 = (c * NS + s) * CT
    r0 = s * ROWS_PER_TILE
    tabs = (tab0_h, tab1_h, tab2_h)
    eis = (ei0_h, ei1_h, ei2_h)
    for e in range(NET):
        tab_h = tabs[e]
        pltpu.sync_copy(eis[e].at[0, pl.ds(w0, CT)], idxs_v)
        pltpu.sync_copy(eis[e].at[1, pl.ds(w0, CT)], idxd_v)
        pltpu.sync_copy(zeros_h.at[pl.ds(r0, ROWS_PER_TILE)],
                        agg_sh.at[pl.ds(r0, ROWS_PER_TILE)])
        plsc.subcore_barrier()
        b = 0
        pltpu.async_copy(tab_h.at[idxs_v.at[b]], rows0_v, sem0)
        pltpu.async_copy(tab_h.at[idxs_v.at[b + 1]], rows1_v, sem1)

        def step(t2, carry):
            t = b + 2 * t2
            pltpu.make_async_copy(tab_h.at[idxs_v.at[t]], rows0_v, sem0).wait()
            pltpu.sync_copy(rows0_v, agg_sh.at[idxd_v.at[t]], add=True)
            pltpu.async_copy(tab_h.at[idxs_v.at[t + 2]], rows0_v, sem0)
            pltpu.make_async_copy(
                tab_h.at[idxs_v.at[t + 1]], rows1_v, sem1).wait()
            pltpu.sync_copy(rows1_v, agg_sh.at[idxd_v.at[t + 1]], add=True)
            pltpu.async_copy(tab_h.at[idxs_v.at[t + 3]], rows1_v, sem1)
            return carry

        lax.fori_loop(0, CT // 2 - 1, step, None)
        t = b + CT - 2
        pltpu.make_async_copy(tab_h.at[idxs_v.at[t]], rows0_v, sem0).wait()
        pltpu.sync_copy(rows0_v, agg_sh.at[idxd_v.at[t]], add=True)
        pltpu.make_async_copy(tab_h.at[idxs_v.at[t + 1]], rows1_v, sem1).wait()
        pltpu.sync_copy(rows1_v, agg_sh.at[idxd_v.at[t + 1]], add=True)
        plsc.subcore_barrier()
        pltpu.sync_copy(agg_sh.at[pl.ds(r0, ROWS_PER_TILE)],
                        out_h.at[e, c, pl.ds(r0, ROWS_PER_TILE)])


def _rs_block(degw):
    return lax.rsqrt(jnp.maximum(degw[0] + degw[1], 1.0))


def _prep_body(x_ref, degw_ref, t0_ref, t1_ref, t2_ref):
    rs = _rs_block(degw_ref[...])
    x = x_ref[...]
    for e, tref in enumerate((t0_ref, t1_ref, t2_ref)):
        tref[...] = x * rs[:, 2 * e:2 * e + 1]


def _prep(x, degw):
    return pl.pallas_call(
        _prep_body,
        grid=(GRID,),
        in_specs=[
            pl.BlockSpec((BT, D), lambda b: (b, 0)),
            pl.BlockSpec((NC, BT, D), lambda b: (0, b, 0)),
        ],
        out_specs=[pl.BlockSpec((BT, D), lambda b: (b, 0))] * NET,
        out_shape=[jax.ShapeDtypeStruct((N, D), jnp.float32)] * NET,
    )(x, degw)


def _layer_a_body(aggs_ref, degw_ref, w3_ref, bsum_ref,
                  fcw_ref, fcb_ref, u_ref, sums_ref):
    b = pl.program_id(0)
    rs = _rs_block(degw_ref[...])
    tot = bsum_ref[...] + jnp.zeros((BT, D), jnp.float32)
    for e in range(NET):
        p = (aggs_ref[e, 0] + aggs_ref[e, 1]) * rs[:, 2 * e + 1:2 * e + 2]
        tot = tot + jnp.dot(p, w3_ref[e], preferred_element_type=jnp.float32)
    u = jnp.dot(tot, fcw_ref[...], preferred_element_type=jnp.float32)
    u = jnp.maximum(u + fcb_ref[...], 0.0)
    u_ref[...] = u

    @pl.when(b == 0)
    def _():
        sums_ref[...] = jnp.zeros_like(sums_ref)

    sums_ref[0:1] = sums_ref[0:1] + jnp.sum(u, axis=0, keepdims=True)
    sums_ref[1:2] = sums_ref[1:2] + jnp.sum(u * u, axis=0, keepdims=True)


def _layer_a(aggs, degw, w3, bsum, fcw, fcb):
    return pl.pallas_call(
        _layer_a_body,
        grid=(GRID,),
        in_specs=[
            pl.BlockSpec((NET, NC, BT, D), lambda b: (0, 0, b, 0)),
            pl.BlockSpec((NC, BT, D), lambda b: (0, b, 0)),
            pl.BlockSpec((NET, D, D), lambda b: (0, 0, 0)),
            pl.BlockSpec((1, D), lambda b: (0, 0)),
            pl.BlockSpec((D, D), lambda b: (0, 0)),
            pl.BlockSpec((1, D), lambda b: (0, 0)),
        ],
        out_specs=[
            pl.BlockSpec((BT, D), lambda b: (b, 0)),
            pl.BlockSpec((2, D), lambda b: (0, 0)),
        ],
        out_shape=[
            jax.ShapeDtypeStruct((N, D), jnp.float32),
            jax.ShapeDtypeStruct((2, D), jnp.float32),
        ],
    )(aggs, degw, w3, bsum, fcw, fcb)


def _norm(u_ref, sums_ref, gb_ref):
    mean = sums_ref[0:1] / N
    var = sums_ref[1:2] / N - mean * mean
    inv = lax.rsqrt(var + 1e-5)
    return (u_ref[...] - mean) * (inv * gb_ref[0:1]) + gb_ref[1:2]


def _layer_b_mid_body(u_ref, sums_ref, gb_ref, degw_ref,
                      t0_ref, t1_ref, t2_ref):
    hn = _norm(u_ref, sums_ref, gb_ref)
    rs = _rs_block(degw_ref[...])
    for e, tref in enumerate((t0_ref, t1_ref, t2_ref)):
        tref[...] = hn * rs[:, 2 * e:2 * e + 1]


def _layer_b_mid(u, sums, gb, degw):
    return pl.pallas_call(
        _layer_b_mid_body,
        grid=(GRID,),
        in_specs=[
            pl.BlockSpec((BT, D), lambda b: (b, 0)),
            pl.BlockSpec((2, D), lambda b: (0, 0)),
            pl.BlockSpec((2, D), lambda b: (0, 0)),
            pl.BlockSpec((NC, BT, D), lambda b: (0, b, 0)),
        ],
        out_specs=[pl.BlockSpec((BT, D), lambda b: (b, 0))] * NET,
        out_shape=[jax.ShapeDtypeStruct((N, D), jnp.float32)] * NET,
    )(u, sums, gb, degw)


def _layer_b_fin_body(u_ref, sums_ref, gb_ref, h_ref):
    h_ref[...] = _norm(u_ref, sums_ref, gb_ref)


def _layer_b_fin(u, sums, gb):
    return pl.pallas_call(
        _layer_b_fin_body,
        grid=(GRID,),
        in_specs=[
            pl.BlockSpec((BT, D), lambda b: (b, 0)),
            pl.BlockSpec((2, D), lambda b: (0, 0)),
            pl.BlockSpec((2, D), lambda b: (0, 0)),
        ],
        out_specs=pl.BlockSpec((BT, D), lambda b: (b, 0)),
        out_shape=jax.ShapeDtypeStruct((N, D), jnp.float32),
    )(u, sums, gb)


def kernel(x, params, edge_index_residue, edge_index_seq, edge_index_knn):
    eis = [ei.astype(jnp.int32) for ei in
           (edge_index_residue, edge_index_seq, edge_index_knn)]
    eic = [ei.reshape(2, NCHUNK, CH) for ei in eis]

    ones_h = jnp.ones((CH,), jnp.float32)
    zeros_deg = jnp.zeros((NSLOT,), jnp.float32)
    pidx = ((jnp.arange(NSLOT, dtype=jnp.int32) * D)[None, :]
            + jnp.arange(2 * NET, dtype=jnp.int32)[:, None]
            ).reshape(2 * NET, 1, NSLOT)
    zeros_agg = jnp.zeros((NPAD, D), jnp.float32)

    degw = _sc_degrees(eic[0], eic[1], eic[2], pidx, ones_h,
                       zeros_deg).reshape(NC, NSLOT, D)
    tabs = _prep(x, degw)

    h = None
    for l, p in enumerate(params):
        w3 = jnp.stack([p['W_residue'], p['W_seq'], p['W_knn']])
        bsum = (p['b_residue'] + p['b_seq'] + p['b_knn']).reshape(1, D)
        fcb = p['fcb'].reshape(1, D)
        gb = jnp.stack([p['gamma'], p['beta']])
        aggs = _sc_agg(tabs[0], tabs[1], tabs[2],
                       eic[0], eic[1], eic[2], zeros_agg)
        u, sums = _layer_a(aggs, degw, w3, bsum, p['fcW'], fcb)
        if l + 1 < len(params):
            tabs = _layer_b_mid(u, sums, gb, degw)
        else:
            h = _layer_b_fin(u, sums, gb)
    return h


# speedup vs baseline: 1.0567x; 1.0016x over previous
import functools

import jax
import jax.numpy as jnp
from jax import lax
from jax.experimental import pallas as pl
from jax.experimental.pallas import tpu as pltpu
from jax.experimental.pallas import tpu_sc as plsc

N = 10000
E = 160000
D = 128
NET = 3

NC = 2
NS = 16
NW = NC * NS

CH = 125
NCHUNK = E // CH
CT = NCHUNK // NW
CORE_ROWS = NCHUNK // NC

NPAD = 10112
ROWS_PER_TILE = NPAD // NS

NSLOT = 10240
NDEGC = 2 * NET * NSLOT
DEGC_PER_TILE = NDEGC // NS
PACK_N = NSLOT // NS
DCHUNK = 2 * NET * NCHUNK
DCT = DCHUNK // NW
EPAD = 163840
ECT = EPAD // NW

BT = 1000
GRID = N // BT

_mesh = plsc.VectorSubcoreMesh(core_axis_name="c", subcore_axis_name="s")


@functools.partial(
    pl.kernel,
    mesh=_mesh,
    out_type=jax.ShapeDtypeStruct((NC * NSLOT * D,), jnp.float32),
    scratch_types=[
        pltpu.VMEM((ECT,), jnp.int32),
        pltpu.VMEM((ECT,), jnp.float32),
        pltpu.VMEM((PACK_N,), jnp.float32),
        pltpu.VMEM((PACK_N,), jnp.int32),
        pltpu.VMEM_SHARED((NSLOT,), jnp.float32),
        pltpu.VMEM_SHARED((NSLOT,), jnp.float32),
        pltpu.VMEM_SHARED((NSLOT,), jnp.float32),
        pltpu.VMEM_SHARED((NSLOT,), jnp.float32),
        pltpu.VMEM_SHARED((NSLOT,), jnp.float32),
        pltpu.VMEM_SHARED((NSLOT,), jnp.float32),
        pltpu.VMEM_SHARED((NSLOT * D,), jnp.float32),
    ],
)
def _sc_degrees(ei0_h, ei1_h, ei2_h, pidx_h, ones_h, zeros_h, out_h,
                idx_v, ones_v, cstage_v, pidx_v,
                dg0, dg1, dg2, dg3, dg4, dg5, pack_sh):
    c = lax.axis_index("c")
    s = lax.axis_index("s")
    eis = (ei0_h, ei1_h, ei2_h)
    degs = (dg0, dg1, dg2, dg3, dg4, dg5)
    w0 = (c * NS + s) * CT
    z0 = s * PACK_N
    pltpu.sync_copy(ones_h, ones_v)
    for d6 in range(2 * NET):
        pltpu.sync_copy(zeros_h.at[pl.ds(z0, PACK_N)],
                        degs[d6].at[pl.ds(z0, PACK_N)])
    plsc.subcore_barrier()
    e0 = (c * NS + s) * ECT
    for d6 in range(2 * NET):
        e, dr = divmod(d6, 2)
        pltpu.sync_copy(eis[e].at[dr, 0, pl.ds(e0, ECT)], idx_v)
        pltpu.sync_copy(ones_v, degs[d6].at[idx_v], add=True)
    plsc.subcore_barrier()
    for d6 in range(2 * NET):
        pltpu.sync_copy(degs[d6].at[pl.ds(z0, PACK_N)], cstage_v)
        pltpu.sync_copy(pidx_h.at[d6, 0, pl.ds(z0, PACK_N)], pidx_v)
        pltpu.sync_copy(cstage_v, pack_sh.at[pidx_v])
    plsc.subcore_barrier()
    r1 = s * PACK_N * D
    pltpu.sync_copy(pack_sh.at[pl.ds(r1, PACK_N * D)],
                    out_h.at[pl.ds(c * NSLOT * D + r1, PACK_N * D)])


@functools.partial(
    pl.kernel,
    mesh=_mesh,
    out_type=jax.ShapeDtypeStruct((NET, NC, NPAD, D), jnp.float32),
    scratch_types=[
        pltpu.VMEM((CT, CH), jnp.int32),
        pltpu.VMEM((CT, CH), jnp.int32),
        pltpu.VMEM((CH, D), jnp.float32),
        pltpu.VMEM((CH, D), jnp.float32),
        pltpu.VMEM_SHARED((NPAD, D), jnp.float32),
        pltpu.SemaphoreType.DMA,
        pltpu.SemaphoreType.DMA,
    ],
)
def _sc_agg(tab0_h, tab1_h, tab2_h, ei0_h, ei1_h, ei2_h, zeros_h, out_h,
            idxs_v, idxd_v, rows0_v, rows1_v, agg_sh, sem0, sem1):
    c = lax.axis_index("c")
    s = lax.axis_index("s")
    w0 = (c * NS + s) * CT
    r0 = s * ROWS_PER_TILE
    tabs = (tab0_h, tab1_h, tab2_h)
    eis = (ei0_h, ei1_h, ei2_h)
    for e in range(NET):
        tab_h = tabs[e]
        pltpu.sync_copy(eis[e].at[0, pl.ds(w0, CT)], idxs_v)
        pltpu.sync_copy(eis[e].at[1, pl.ds(w0, CT)], idxd_v)
        pltpu.sync_copy(zeros_h.at[pl.ds(r0, ROWS_PER_TILE)],
                        agg_sh.at[pl.ds(r0, ROWS_PER_TILE)])
        plsc.subcore_barrier()
        b = 0
        pltpu.async_copy(tab_h.at[idxs_v.at[b]], rows0_v, sem0)
        pltpu.async_copy(tab_h.at[idxs_v.at[b + 1]], rows1_v, sem1)

        def step(t2, carry):
            t = b + 2 * t2
            pltpu.make_async_copy(tab_h.at[idxs_v.at[t]], rows0_v, sem0).wait()
            pltpu.sync_copy(rows0_v, agg_sh.at[idxd_v.at[t]], add=True)
            pltpu.async_copy(tab_h.at[idxs_v.at[t + 2]], rows0_v, sem0)
            pltpu.make_async_copy(
                tab_h.at[idxs_v.at[t + 1]], rows1_v, sem1).wait()
            pltpu.sync_copy(rows1_v, agg_sh.at[idxd_v.at[t + 1]], add=True)
            pltpu.async_copy(tab_h.at[idxs_v.at[t + 3]], rows1_v, sem1)
            return carry

        lax.fori_loop(0, CT // 2 - 1, step, None)
        t = b + CT - 2
        pltpu.make_async_copy(tab_h.at[idxs_v.at[t]], rows0_v, sem0).wait()
        pltpu.sync_copy(rows0_v, agg_sh.at[idxd_v.at[t]], add=True)
        pltpu.make_async_copy(tab_h.at[idxs_v.at[t + 1]], rows1_v, sem1).wait()
        pltpu.sync_copy(rows1_v, agg_sh.at[idxd_v.at[t + 1]], add=True)
        plsc.subcore_barrier()
        pltpu.sync_copy(agg_sh.at[pl.ds(r0, ROWS_PER_TILE)],
                        out_h.at[e, c, pl.ds(r0, ROWS_PER_TILE)])


def _rs_block(degw):
    return lax.rsqrt(jnp.maximum(degw[0] + degw[1], 1.0))


def _prep_body(x_ref, degw_ref, t0_ref, t1_ref, t2_ref):
    rs = _rs_block(degw_ref[...])
    x = x_ref[...]
    for e, tref in enumerate((t0_ref, t1_ref, t2_ref)):
        tref[...] = x * rs[:, 2 * e:2 * e + 1]


def _prep(x, degw):
    return pl.pallas_call(
        _prep_body,
        grid=(GRID,),
        in_specs=[
            pl.BlockSpec((BT, D), lambda b: (b, 0)),
            pl.BlockSpec((NC, BT, D), lambda b: (0, b, 0)),
        ],
        out_specs=[pl.BlockSpec((BT, D), lambda b: (b, 0))] * NET,
        out_shape=[jax.ShapeDtypeStruct((N, D), jnp.float32)] * NET,
    )(x, degw)


def _layer_a_body(aggs_ref, degw_ref, w3_ref, bsum_ref,
                  fcw_ref, fcb_ref, u_ref, sums_ref):
    b = pl.program_id(0)
    rs = _rs_block(degw_ref[...])
    tot = bsum_ref[...] + jnp.zeros((BT, D), jnp.float32)
    for e in range(NET):
        p = (aggs_ref[e, 0] + aggs_ref[e, 1]) * rs[:, 2 * e + 1:2 * e + 2]
        tot = tot + jnp.dot(p, w3_ref[e], preferred_element_type=jnp.float32)
    u = jnp.dot(tot, fcw_ref[...], preferred_element_type=jnp.float32)
    u = jnp.maximum(u + fcb_ref[...], 0.0)
    u_ref[...] = u

    @pl.when(b == 0)
    def _():
        sums_ref[...] = jnp.zeros_like(sums_ref)

    sums_ref[0:1] = sums_ref[0:1] + jnp.sum(u, axis=0, keepdims=True)
    sums_ref[1:2] = sums_ref[1:2] + jnp.sum(u * u, axis=0, keepdims=True)


def _layer_a(aggs, degw, w3, bsum, fcw, fcb):
    return pl.pallas_call(
        _layer_a_body,
        grid=(GRID,),
        in_specs=[
            pl.BlockSpec((NET, NC, BT, D), lambda b: (0, 0, b, 0)),
            pl.BlockSpec((NC, BT, D), lambda b: (0, b, 0)),
            pl.BlockSpec((NET, D, D), lambda b: (0, 0, 0)),
            pl.BlockSpec((1, D), lambda b: (0, 0)),
            pl.BlockSpec((D, D), lambda b: (0, 0)),
            pl.BlockSpec((1, D), lambda b: (0, 0)),
        ],
        out_specs=[
            pl.BlockSpec((BT, D), lambda b: (b, 0)),
            pl.BlockSpec((2, D), lambda b: (0, 0)),
        ],
        out_shape=[
            jax.ShapeDtypeStruct((N, D), jnp.float32),
            jax.ShapeDtypeStruct((2, D), jnp.float32),
        ],
    )(aggs, degw, w3, bsum, fcw, fcb)


def _norm(u_ref, sums_ref, gb_ref):
    mean = sums_ref[0:1] / N
    var = sums_ref[1:2] / N - mean * mean
    inv = lax.rsqrt(var + 1e-5)
    return (u_ref[...] - mean) * (inv * gb_ref[0:1]) + gb_ref[1:2]


def _layer_b_mid_body(u_ref, sums_ref, gb_ref, degw_ref,
                      t0_ref, t1_ref, t2_ref):
    hn = _norm(u_ref, sums_ref, gb_ref)
    rs = _rs_block(degw_ref[...])
    for e, tref in enumerate((t0_ref, t1_ref, t2_ref)):
        tref[...] = hn * rs[:, 2 * e:2 * e + 1]


def _layer_b_mid(u, sums, gb, degw):
    return pl.pallas_call(
        _layer_b_mid_body,
        grid=(GRID,),
        in_specs=[
            pl.BlockSpec((BT, D), lambda b: (b, 0)),
            pl.BlockSpec((2, D), lambda b: (0, 0)),
            pl.BlockSpec((2, D), lambda b: (0, 0)),
            pl.BlockSpec((NC, BT, D), lambda b: (0, b, 0)),
        ],
        out_specs=[pl.BlockSpec((BT, D), lambda b: (b, 0))] * NET,
        out_shape=[jax.ShapeDtypeStruct((N, D), jnp.float32)] * NET,
    )(u, sums, gb, degw)


def _layer_b_fin_body(u_ref, sums_ref, gb_ref, h_ref):
    h_ref[...] = _norm(u_ref, sums_ref, gb_ref)


def _layer_b_fin(u, sums, gb):
    return pl.pallas_call(
        _layer_b_fin_body,
        grid=(GRID,),
        in_specs=[
            pl.BlockSpec((BT, D), lambda b: (b, 0)),
            pl.BlockSpec((2, D), lambda b: (0, 0)),
            pl.BlockSpec((2, D), lambda b: (0, 0)),
        ],
        out_specs=pl.BlockSpec((BT, D), lambda b: (b, 0)),
        out_shape=jax.ShapeDtypeStruct((N, D), jnp.float32),
    )(u, sums, gb)


def kernel(x, params, edge_index_residue, edge_index_seq, edge_index_knn):
    eis = [ei.astype(jnp.int32) for ei in
           (edge_index_residue, edge_index_seq, edge_index_knn)]
    eic = [ei.reshape(2, NCHUNK, CH) for ei in eis]

    ones_h = jnp.ones((ECT,), jnp.float32)
    zeros_deg = jnp.zeros((NSLOT,), jnp.float32)
    pidx = ((jnp.arange(NSLOT, dtype=jnp.int32) * D)[None, :]
            + jnp.arange(2 * NET, dtype=jnp.int32)[:, None]
            ).reshape(2 * NET, 1, NSLOT)
    zeros_agg = jnp.zeros((NPAD, D), jnp.float32)

    pad = jnp.full((2, EPAD - E), N, jnp.int32)
    eif = [jnp.concatenate([ei, pad], axis=1).reshape(2, 1, EPAD)
           for ei in eis]
    degw = _sc_degrees(eif[0], eif[1], eif[2], pidx, ones_h,
                       zeros_deg).reshape(NC, NSLOT, D)
    tabs = _prep(x, degw)

    h = None
    for l, p in enumerate(params):
        w3 = jnp.stack([p['W_residue'], p['W_seq'], p['W_knn']])
        bsum = (p['b_residue'] + p['b_seq'] + p['b_knn']).reshape(1, D)
        fcb = p['fcb'].reshape(1, D)
        gb = jnp.stack([p['gamma'], p['beta']])
        aggs = _sc_agg(tabs[0], tabs[1], tabs[2],
                       eic[0], eic[1], eic[2], zeros_agg)
        u, sums = _layer_a(aggs, degw, w3, bsum, p['fcW'], fcb)
        if l + 1 < len(params):
            tabs = _layer_b_mid(u, sums, gb, degw)
        else:
            h = _layer_b_fin(u, sums, gb)
    return h
